# baseline (speedup 1.0000x reference)
import functools

import jax
import jax.numpy as jnp
from jax import lax
from jax.experimental import pallas as pl
from jax.experimental.pallas import tpu as pltpu
from jax.experimental.pallas import tpu_sc as plsc

_GAMMA = 12.0
_LANES = 16


def _make_score_kernel(batch, hidden):
    info = plsc.get_sparse_core_info()
    nc, ns = info.num_cores, info.num_subcores
    nw = nc * ns
    assert batch % (2 * nw * _LANES) == 0 and hidden % _LANES == 0
    bpw = batch // nw
    half = bpw // 2
    ngrp = bpw // _LANES

    @functools.partial(
        pl.kernel,
        mesh=plsc.VectorSubcoreMesh(core_axis_name="c", subcore_axis_name="s",
                                    num_cores=nc, num_subcores=ns),
        out_type=jax.ShapeDtypeStruct((batch,), jnp.float32),
        compiler_params=pltpu.CompilerParams(needs_layout_passes=False),
        scratch_types=[
            pltpu.VMEM((2, 2 * half), jnp.int32),
            pltpu.VMEM((2, half), jnp.int32),
            pltpu.VMEM((2, 2 * half, hidden), jnp.float32),
            pltpu.VMEM((2, half, hidden), jnp.float32),
            pltpu.VMEM((bpw, _LANES), jnp.float32),
            pltpu.VMEM((bpw,), jnp.float32),
            pltpu.SemaphoreType.DMA,
            pltpu.SemaphoreType.DMA,
            pltpu.SemaphoreType.DMA,
            pltpu.SemaphoreType.DMA,
            pltpu.SemaphoreType.DMA,
        ],
    )
    def score(h_idx_hbm, r_idx_hbm, t_idx_hbm, ent_hbm, rel_hbm, out_hbm,
              ht_idx_v, r_idx_v, ht_v, r_v, acc_v, out_v,
              sem_i, sem_e0, sem_r0, sem_e1, sem_r1):
        wid = lax.axis_index("s") * nc + lax.axis_index("c")
        base = wid * bpw
        lane_ids = lax.iota(jnp.int32, _LANES)

        def fire(b, sem_e, sem_r):
            off = base + b * half
            i0 = pltpu.async_copy(h_idx_hbm.at[pl.ds(off, half)],
                                  ht_idx_v.at[b, pl.ds(0, half)], sem_i)
            i1 = pltpu.async_copy(t_idx_hbm.at[pl.ds(off, half)],
                                  ht_idx_v.at[b, pl.ds(half, half)], sem_i)
            i2 = pltpu.async_copy(r_idx_hbm.at[pl.ds(off, half)],
                                  r_idx_v.at[b], sem_i)
            i0.wait()
            i1.wait()
            ce = pltpu.async_copy(ent_hbm.at[ht_idx_v.at[b]], ht_v.at[b], sem_e)
            i2.wait()
            cr = pltpu.async_copy(rel_hbm.at[r_idx_v.at[b]], r_v.at[b], sem_r)
            return ce, cr

        def compute(b):
            @plsc.parallel_loop(0, half, 1, unroll=1)
            def _stage1(i):
                acc = jnp.zeros((_LANES,), jnp.float32)
                for d in range(hidden // _LANES):
                    sl = pl.ds(d * _LANES, _LANES)
                    acc = acc + jnp.abs(ht_v[b, i, sl] + (r_v[b, i, sl] - ht_v[b, half + i, sl]))
                acc_v[b * half + i] = acc

        ce0, cr0 = fire(0, sem_e0, sem_r0)
        ce1, cr1 = fire(1, sem_e1, sem_r1)
        ce0.wait()
        cr0.wait()
        compute(0)
        ce1.wait()
        cr1.wait()
        compute(1)

        @plsc.parallel_loop(0, ngrp, 1, unroll=1)
        def _stage2(g):
            rows = lane_ids + g * _LANES
            tot = jnp.zeros((_LANES,), jnp.float32)
            for j in range(_LANES):
                cols = jnp.full((_LANES,), j, jnp.int32)
                tot = tot + plsc.load_gather(acc_v, [rows, cols])
            out_v[pl.ds(g * _LANES, _LANES)] = _GAMMA - tot
        pltpu.sync_copy(out_v, out_hbm.at[pl.ds(base, bpw)])

    return score


def kernel(positive_sample, negative_sample, mode, entity_embedding, relation_embedding):
    del negative_sample, mode
    batch = positive_sample.shape[0]
    hidden = entity_embedding.shape[1]
    h_idx = positive_sample[:, 0]
    r_idx = positive_sample[:, 1]
    t_idx = positive_sample[:, 2]
    score = _make_score_kernel(batch, hidden)
    out = score(h_idx, r_idx, t_idx, entity_embedding, relation_embedding)
    return out.reshape(batch, 1)


# speedup vs baseline: 1.0218x; 1.0218x over previous
import functools

import jax
import jax.numpy as jnp
from jax import lax
from jax.experimental import pallas as pl
from jax.experimental.pallas import tpu as pltpu
from jax.experimental.pallas import tpu_sc as plsc

_GAMMA = 12.0
_LANES = 16


def _make_score_kernel(batch, hidden):
    info = plsc.get_sparse_core_info()
    nc, ns = 1, info.num_subcores
    nw = nc * ns
    assert batch % (2 * nw * _LANES) == 0 and hidden % _LANES == 0
    bpw = batch // nw
    half = bpw // 2
    ngrp = bpw // _LANES

    @functools.partial(
        pl.kernel,
        mesh=plsc.VectorSubcoreMesh(core_axis_name="c", subcore_axis_name="s",
                                    num_cores=nc, num_subcores=ns),
        out_type=jax.ShapeDtypeStruct((batch,), jnp.float32),
        compiler_params=pltpu.CompilerParams(needs_layout_passes=False),
        scratch_types=[
            pltpu.VMEM((2, 2 * half), jnp.int32),
            pltpu.VMEM((2, half), jnp.int32),
            pltpu.VMEM((2, 2 * half, hidden), jnp.float32),
            pltpu.VMEM((2, half, hidden), jnp.float32),
            pltpu.VMEM((bpw, _LANES), jnp.float32),
            pltpu.VMEM((bpw,), jnp.float32),
            pltpu.SemaphoreType.DMA,
            pltpu.SemaphoreType.DMA,
            pltpu.SemaphoreType.DMA,
            pltpu.SemaphoreType.DMA,
            pltpu.SemaphoreType.DMA,
        ],
    )
    def score(h_idx_hbm, r_idx_hbm, t_idx_hbm, ent_hbm, rel_hbm, out_hbm,
              ht_idx_v, r_idx_v, ht_v, r_v, acc_v, out_v,
              sem_i, sem_e0, sem_r0, sem_e1, sem_r1):
        wid = lax.axis_index("s") * nc + lax.axis_index("c")
        base = wid * bpw
        lane_ids = lax.iota(jnp.int32, _LANES)

        def fire(b, sem_e, sem_r):
            off = base + b * half
            i0 = pltpu.async_copy(h_idx_hbm.at[pl.ds(off, half)],
                                  ht_idx_v.at[b, pl.ds(0, half)], sem_i)
            i1 = pltpu.async_copy(t_idx_hbm.at[pl.ds(off, half)],
                                  ht_idx_v.at[b, pl.ds(half, half)], sem_i)
            i2 = pltpu.async_copy(r_idx_hbm.at[pl.ds(off, half)],
                                  r_idx_v.at[b], sem_i)
            i0.wait()
            i1.wait()
            ce = pltpu.async_copy(ent_hbm.at[ht_idx_v.at[b]], ht_v.at[b], sem_e)
            i2.wait()
            cr = pltpu.async_copy(rel_hbm.at[r_idx_v.at[b]], r_v.at[b], sem_r)
            return ce, cr

        def compute(b):
            @plsc.parallel_loop(0, half, 1, unroll=1)
            def _stage1(i):
                acc = jnp.zeros((_LANES,), jnp.float32)
                for d in range(hidden // _LANES):
                    sl = pl.ds(d * _LANES, _LANES)
                    acc = acc + jnp.abs(ht_v[b, i, sl] + (r_v[b, i, sl] - ht_v[b, half + i, sl]))
                acc_v[b * half + i] = acc

        ce0, cr0 = fire(0, sem_e0, sem_r0)
        ce1, cr1 = fire(1, sem_e1, sem_r1)
        ce0.wait()
        cr0.wait()
        compute(0)
        ce1.wait()
        cr1.wait()
        compute(1)

        @plsc.parallel_loop(0, ngrp, 1, unroll=1)
        def _stage2(g):
            rows = lane_ids + g * _LANES
            tot = jnp.zeros((_LANES,), jnp.float32)
            for j in range(_LANES):
                cols = jnp.full((_LANES,), j, jnp.int32)
                tot = tot + plsc.load_gather(acc_v, [rows, cols])
            out_v[pl.ds(g * _LANES, _LANES)] = _GAMMA - tot
        pltpu.sync_copy(out_v, out_hbm.at[pl.ds(base, bpw)])

    return score


def kernel(positive_sample, negative_sample, mode, entity_embedding, relation_embedding):
    del negative_sample, mode
    batch = positive_sample.shape[0]
    hidden = entity_embedding.shape[1]
    h_idx = positive_sample[:, 0]
    r_idx = positive_sample[:, 1]
    t_idx = positive_sample[:, 2]
    score = _make_score_kernel(batch, hidden)
    out = score(h_idx, r_idx, t_idx, entity_embedding, relation_embedding)
    return out.reshape(batch, 1)
